# baseline (speedup 1.0000x reference)
import functools

import jax
import jax.numpy as jnp
from jax import lax
from jax.experimental import pallas as pl
from jax.experimental.pallas import tpu as pltpu
from jax.experimental.pallas import tpu_sc as plsc

_CHUNK = 2
_GROUPS = 2


@functools.partial(jax.jit, static_argnums=(2,))
def _sc_embedding_lookup(table, idx3d, rows):
    hidden = table.shape[1]
    num_w = idx3d.shape[0]
    mesh = plsc.VectorSubcoreMesh(
        core_axis_name="c", subcore_axis_name="s", num_cores=1
    )

    @functools.partial(
        pl.kernel,
        mesh=mesh,
        out_type=jax.ShapeDtypeStruct((rows, hidden), jnp.float32),
        scratch_types=[
            pltpu.VMEM((_CHUNK,), jnp.int32),
            pltpu.VMEM((_CHUNK,), jnp.int32),
            pltpu.VMEM((_CHUNK, hidden), jnp.float32),
            pltpu.VMEM((_CHUNK, hidden), jnp.float32),
            pltpu.SemaphoreType.DMA,
            pltpu.SemaphoreType.DMA,
            pltpu.SemaphoreType.DMA,
            pltpu.SemaphoreType.DMA,
            pltpu.SemaphoreType.DMA,
        ],
    )
    def gather_kernel(
        table_hbm, idx_hbm, out_hbm,
        idx_v0, idx_v1, buf0, buf1, sem_i0, sem_i1, sem_g0, sem_g1, sem_w,
    ):
        wid = lax.axis_index("s")

        @pl.when(wid < num_w)
        def _():
            base = wid * (_CHUNK * _GROUPS)
            a0 = pltpu.async_copy(idx_hbm.at[wid, 0], idx_v0, sem_i0)
            a1 = pltpu.async_copy(idx_hbm.at[wid, 1], idx_v1, sem_i1)
            a0.wait()
            g0 = pltpu.async_copy(table_hbm.at[idx_v0], buf0, sem_g0)
            a1.wait()
            g1 = pltpu.async_copy(table_hbm.at[idx_v1], buf1, sem_g1)
            g0.wait()
            w0 = pltpu.async_copy(buf0, out_hbm.at[pl.ds(base, _CHUNK)], sem_w)
            g1.wait()
            w1 = pltpu.async_copy(
                buf1, out_hbm.at[pl.ds(base + _CHUNK, _CHUNK)], sem_w
            )
            w0.wait()
            w1.wait()

    return gather_kernel(table, idx3d)


def kernel(embedding_weight, seq_indices):
    rows = seq_indices.shape[0]
    idx3d = jnp.asarray(seq_indices, jnp.int32).reshape(-1, _GROUPS, _CHUNK)
    return _sc_embedding_lookup(embedding_weight, idx3d, rows)


# speedup vs baseline: 1.0027x; 1.0027x over previous
import functools

import jax
import jax.numpy as jnp
from jax import lax
from jax.experimental import pallas as pl
from jax.experimental.pallas import tpu as pltpu
from jax.experimental.pallas import tpu_sc as plsc

_CHUNK = 4


@functools.partial(jax.jit, static_argnums=(2,))
def _sc_embedding_lookup(table, idx2d, rows):
    hidden = table.shape[1]
    num_w = idx2d.shape[0]
    mesh = plsc.VectorSubcoreMesh(
        core_axis_name="c", subcore_axis_name="s", num_cores=1
    )

    @functools.partial(
        pl.kernel,
        mesh=mesh,
        out_type=jax.ShapeDtypeStruct((rows, hidden), jnp.float32),
        scratch_types=[
            pltpu.VMEM((_CHUNK,), jnp.int32),
            pltpu.VMEM((_CHUNK, hidden), jnp.float32),
            pltpu.SemaphoreType.DMA,
        ],
    )
    def gather_kernel(table_hbm, idx_hbm, out_hbm, idx_v, rows_v, sem):
        wid = lax.axis_index("s")

        @pl.when(wid < num_w)
        def _():
            pltpu.sync_copy(idx_hbm.at[wid], idx_v)
            pltpu.async_copy(table_hbm.at[idx_v], rows_v, sem).wait()
            pltpu.sync_copy(rows_v, out_hbm.at[pl.ds(wid * _CHUNK, _CHUNK)])

    return gather_kernel(table, idx2d)


def kernel(embedding_weight, seq_indices):
    rows = seq_indices.shape[0]
    idx2d = jnp.asarray(seq_indices, jnp.int32).reshape(-1, _CHUNK)
    return _sc_embedding_lookup(embedding_weight, idx2d, rows)
